# baseline (speedup 1.0000x reference)
import functools

import jax
import jax.numpy as jnp
import numpy as np
from jax import lax
from jax.experimental import pallas as pl
from jax.experimental.pallas import tpu as pltpu
from jax.experimental.pallas import tpu_sc as plsc

F32 = jnp.float32
_H = 128
_SC_NC = 2
_SC_NS = 16
_NW = _SC_NC * _SC_NS

_ILV = np.empty(_H, np.int32)
_ILV[0::2] = np.arange(_H // 2)
_ILV[1::2] = np.arange(_H // 2, _H)


def _pack32(t):
    return lax.bitcast_convert_type(
        t.reshape(t.shape[0], t.shape[1] // 2, 2), jnp.uint32)


def _unpack32(w):
    lo = lax.bitcast_convert_type(w << 16, F32)
    hi = lax.bitcast_convert_type(w & jnp.uint32(0xFFFF0000), F32)
    return lo, hi


def _ln_rows(t, g, b, eps=1e-5):
    m = jnp.mean(t, axis=-1, keepdims=True)
    v = jnp.mean((t - m) * (t - m), axis=-1, keepdims=True)
    return (t - m) * lax.rsqrt(v + eps) * g + b


def _mlp2_body(x_ref, w1_ref, b1_ref, g_ref, bb_ref, w2_ref, b2_ref, o_ref,
               *, bf16_dot2):
    t = jnp.dot(x_ref[...], w1_ref[...], preferred_element_type=F32) + b1_ref[...]
    t = jnp.maximum(t, 0.0)
    t = _ln_rows(t, g_ref[...], bb_ref[...])
    if bf16_dot2:
        o = jnp.dot(t.astype(jnp.bfloat16), w2_ref[...].astype(jnp.bfloat16),
                    preferred_element_type=F32) + b2_ref[...]
    else:
        o = jnp.dot(t, w2_ref[...], preferred_element_type=F32) + b2_ref[...]
    o_ref[...] = o.astype(o_ref.dtype)


def _mlp2(x, w1t, b1, g, bb, w2t, b2, blk, bf16_dot2=False, out_dtype=F32):
    n, fin = x.shape
    grid = (pl.cdiv(n, blk),)
    return pl.pallas_call(
        functools.partial(_mlp2_body, bf16_dot2=bf16_dot2),
        grid=grid,
        in_specs=[
            pl.BlockSpec((blk, fin), lambda i: (i, 0)),
            pl.BlockSpec((fin, _H), lambda i: (0, 0)),
            pl.BlockSpec((1, _H), lambda i: (0, 0)),
            pl.BlockSpec((1, _H), lambda i: (0, 0)),
            pl.BlockSpec((1, _H), lambda i: (0, 0)),
            pl.BlockSpec((_H, _H), lambda i: (0, 0)),
            pl.BlockSpec((1, _H), lambda i: (0, 0)),
        ],
        out_specs=pl.BlockSpec((blk, _H), lambda i: (i, 0)),
        out_shape=jax.ShapeDtypeStruct((n, _H), out_dtype),
        compiler_params=pltpu.CompilerParams(dimension_semantics=("parallel",)),
    )(x, w1t, b1, g, bb, w2t, b2)


def _node_linear_body(h_ref, w_ref, b_ref, a1_ref, t_ref):
    hw = jnp.dot(h_ref[...], w_ref[...], preferred_element_type=F32) + b_ref[...]
    a1_ref[...] = hw[:, 0 * _H:1 * _H]
    t_ref[...] = hw[:, 1 * _H:5 * _H].astype(jnp.bfloat16)


def _node_linear(h, w, b, blk):
    n = h.shape[0]
    grid = (pl.cdiv(n, blk),)
    return pl.pallas_call(
        _node_linear_body,
        grid=grid,
        in_specs=[
            pl.BlockSpec((blk, _H), lambda i: (i, 0)),
            pl.BlockSpec((_H, 5 * _H), lambda i: (0, 0)),
            pl.BlockSpec((1, 5 * _H), lambda i: (0, 0)),
        ],
        out_specs=[
            pl.BlockSpec((blk, _H), lambda i: (i, 0)),
            pl.BlockSpec((blk, 4 * _H), lambda i: (i, 0)),
        ],
        out_shape=[
            jax.ShapeDtypeStruct((n, _H), F32),
            jax.ShapeDtypeStruct((n, 4 * _H), jnp.bfloat16),
        ],
        compiler_params=pltpu.CompilerParams(dimension_semantics=("parallel",)),
    )(h, w, b)


def _edge_layer_body(e_ref, ur_ref, vc_ref, w_ref, wb_ref, g_ref, b_ref,
                     eo_ref, m_ref):
    e = e_ref[...].astype(F32)
    q = _H // 2
    ulo, uhi = _unpack32(ur_ref[...])
    vlo, vhi = _unpack32(vc_ref[...])

    def piece(lo, hi, gidx):
        return jnp.concatenate(
            [lo[:, gidx * q:(gidx + 1) * q], hi[:, gidx * q:(gidx + 1) * q]],
            axis=1)

    a2r = piece(ulo, uhi, 0)
    b2r = piece(ulo, uhi, 2)
    b3r = piece(ulo, uhi, 3)
    a3c = piece(vlo, vhi, 1)
    b2c = piece(vlo, vhi, 2)
    b3c = piece(vlo, vhi, 3)
    b1h = jnp.dot(e_ref[...], w_ref[...].astype(jnp.bfloat16),
                  preferred_element_type=F32) + wb_ref[...]
    g = g_ref[...]
    b = b_ref[...]

    def gate(t):
        t = jnp.maximum(t, 0.0)
        t = _ln_rows(t, g, b)
        t = e + t
        return t, jax.nn.sigmoid(t)

    e_ji, s_ji = gate(b1h + b2r + b3c)
    e_ik, s_ik = gate(b1h + b2c + b3r)
    m_ji = a2r * s_ji * (1.0 / (jnp.sum(s_ji, axis=1, keepdims=True) + 1e-6))
    m_ik = a3c * s_ik * (1.0 / (jnp.sum(s_ik, axis=1, keepdims=True) + 1e-6))
    eo_ref[...] = e_ji.astype(jnp.bfloat16)
    m_ref[0, :, :] = m_ji
    m_ref[1, :, :] = m_ik


def _edge_layer(e, ur, vc, w, wb, g, b, blk):
    n = e.shape[0]
    grid = (pl.cdiv(n, blk),)
    return pl.pallas_call(
        _edge_layer_body,
        grid=grid,
        in_specs=[
            pl.BlockSpec((blk, _H), lambda i: (i, 0)),
            pl.BlockSpec((blk, 2 * _H), lambda i: (i, 0)),
            pl.BlockSpec((blk, 2 * _H), lambda i: (i, 0)),
            pl.BlockSpec((_H, _H), lambda i: (0, 0)),
            pl.BlockSpec((1, _H), lambda i: (0, 0)),
            pl.BlockSpec((1, _H), lambda i: (0, 0)),
            pl.BlockSpec((1, _H), lambda i: (0, 0)),
        ],
        out_specs=[
            pl.BlockSpec((blk, _H), lambda i: (i, 0)),
            pl.BlockSpec((2, blk, _H), lambda i: (0, i, 0)),
        ],
        out_shape=[
            jax.ShapeDtypeStruct((n, _H), jnp.bfloat16),
            jax.ShapeDtypeStruct((2, n, _H), F32),
        ],
        compiler_params=pltpu.CompilerParams(dimension_semantics=("parallel",)),
    )(e, ur, vc, w, wb, g, b)


def _node_update_body(h_ref, a1_ref, acca_ref, accb_ref, g_ref, b_ref, o_ref):
    t = (a1_ref[...] + acca_ref[0, :, :] + acca_ref[1, :, :]
         + accb_ref[0, :, :] + accb_ref[1, :, :])
    t = jnp.maximum(t, 0.0)
    t = _ln_rows(t, g_ref[...], b_ref[...])
    o_ref[...] = h_ref[...] + t


def _node_update(h, a1h, acca, accb, g, b, blk):
    n = h.shape[0]
    grid = (pl.cdiv(n, blk),)
    return pl.pallas_call(
        _node_update_body,
        grid=grid,
        in_specs=[
            pl.BlockSpec((blk, _H), lambda i: (i, 0)),
            pl.BlockSpec((blk, _H), lambda i: (i, 0)),
            pl.BlockSpec((2, blk, _H), lambda i: (0, i, 0)),
            pl.BlockSpec((2, blk, _H), lambda i: (0, i, 0)),
            pl.BlockSpec((1, _H), lambda i: (0, 0)),
            pl.BlockSpec((1, _H), lambda i: (0, 0)),
        ],
        out_specs=pl.BlockSpec((blk, _H), lambda i: (i, 0)),
        out_shape=jax.ShapeDtypeStruct((n, _H), F32),
        compiler_params=pltpu.CompilerParams(dimension_semantics=("parallel",)),
    )(h, a1h, acca, accb, g, b)


def _score_body(hr_ref, hc_ref, e_ref, wa_ref, wb_ref, wc_ref, b1_ref,
                w2_ref, b2_ref, o_ref):
    bf = jnp.bfloat16
    t = (jnp.dot(hr_ref[...].astype(bf), wa_ref[...].astype(bf),
                 preferred_element_type=F32)
         + jnp.dot(hc_ref[...].astype(bf), wb_ref[...].astype(bf),
                   preferred_element_type=F32)
         + jnp.dot(e_ref[...].astype(bf), wc_ref[...].astype(bf),
                   preferred_element_type=F32)
         + b1_ref[...])
    t = jnp.maximum(t, 0.0)
    o_ref[...] = jnp.dot(t, w2_ref[...], preferred_element_type=F32) + b2_ref[...]


def _score(hr, hc, e, wa, wb, wc, b1, w2, b2, blk):
    n = hr.shape[0]
    grid = (pl.cdiv(n, blk),)
    return pl.pallas_call(
        _score_body,
        grid=grid,
        in_specs=[
            pl.BlockSpec((blk, _H), lambda i: (i, 0)),
            pl.BlockSpec((blk, _H), lambda i: (i, 0)),
            pl.BlockSpec((blk, _H), lambda i: (i, 0)),
            pl.BlockSpec((_H, _H), lambda i: (0, 0)),
            pl.BlockSpec((_H, _H), lambda i: (0, 0)),
            pl.BlockSpec((_H, _H), lambda i: (0, 0)),
            pl.BlockSpec((1, _H), lambda i: (0, 0)),
            pl.BlockSpec((_H, 1), lambda i: (0, 0)),
            pl.BlockSpec((1, 1), lambda i: (0, 0)),
        ],
        out_specs=pl.BlockSpec((blk, 1), lambda i: (i, 0)),
        out_shape=jax.ShapeDtypeStruct((n, 1), F32),
        compiler_params=pltpu.CompilerParams(dimension_semantics=("parallel",)),
    )(hr, hc, e, wa, wb, wc, b1, w2, b2)


def _make_gather2(n_rows, d, e_total, chunk, dtype=F32):
    epw = e_total // _NW
    nch = epw // chunk
    mesh = plsc.VectorSubcoreMesh(core_axis_name="c", subcore_axis_name="s")

    @functools.partial(
        pl.kernel,
        mesh=mesh,
        out_type=[jax.ShapeDtypeStruct((e_total, d), dtype),
                  jax.ShapeDtypeStruct((e_total, d), dtype)],
        scratch_types=[
            pltpu.VMEM((epw,), jnp.int32),
            pltpu.VMEM((epw,), jnp.int32),
            pltpu.VMEM((2, chunk, d), dtype),
            pltpu.VMEM((2, chunk, d), dtype),
            pltpu.SemaphoreType.DMA,
            pltpu.SemaphoreType.DMA,
            pltpu.SemaphoreType.DMA,
            pltpu.SemaphoreType.DMA,
        ],
    )
    def gk(t0_hbm, t1_hbm, i0_hbm, i1_hbm, o0_hbm, o1_hbm,
           i0v, i1v, b0, b1, sg0, sg1, sw0, sw1):
        wid = lax.axis_index("s") * _SC_NC + lax.axis_index("c")
        base = wid * epw

        def gwait(p):
            pltpu.make_async_copy(t0_hbm.at[pl.ds(0, chunk)], b0.at[p], sg0).wait()
            pltpu.make_async_copy(t1_hbm.at[pl.ds(0, chunk)], b1.at[p], sg1).wait()

        def wwait(p):
            pltpu.make_async_copy(b0.at[p], o0_hbm.at[pl.ds(0, chunk)], sw0).wait()
            pltpu.make_async_copy(b1.at[p], o1_hbm.at[pl.ds(0, chunk)], sw1).wait()

        def gissue(g, p):
            pltpu.async_copy(t0_hbm.at[i0v.at[pl.ds(g * chunk, chunk)]],
                             b0.at[p], sg0)
            pltpu.async_copy(t1_hbm.at[i1v.at[pl.ds(g * chunk, chunk)]],
                             b1.at[p], sg1)

        pltpu.sync_copy(i0_hbm.at[pl.ds(base, epw)], i0v)
        pltpu.sync_copy(i1_hbm.at[pl.ds(base, epw)], i1v)
        gissue(0, 0)

        def body(g, carry):
            p = lax.rem(g, 2)
            pn = 1 - p
            off = base + g * chunk
            gwait(p)
            pltpu.async_copy(b0.at[p], o0_hbm.at[pl.ds(off, chunk)], sw0)
            pltpu.async_copy(b1.at[p], o1_hbm.at[pl.ds(off, chunk)], sw1)

            @pl.when(g + 1 < nch)
            def _():
                @pl.when(g >= 1)
                def _():
                    wwait(pn)

                gissue(g + 1, pn)
            return carry

        lax.fori_loop(0, nch, body, 0)
        wwait(0)
        wwait(1)

    return gk


def _make_scatter2(n_rows, d, e_total, chunk):
    ept = e_total // _SC_NS
    nch = ept // chunk
    rpt = n_rows // _SC_NS
    assert rpt % chunk == 0 and chunk % 8 == 0 and n_rows % _SC_NS == 0
    rch = rpt // chunk
    mesh = plsc.VectorSubcoreMesh(core_axis_name="c", subcore_axis_name="s")

    @functools.partial(
        pl.kernel,
        mesh=mesh,
        out_type=jax.ShapeDtypeStruct((2, n_rows, d), F32),
        scratch_types=[
            pltpu.VMEM((2, chunk), jnp.int32),
            pltpu.VMEM((2, chunk, d), F32),
            pltpu.VMEM_SHARED((n_rows, d), F32),
            pltpu.SemaphoreType.DMA,
            pltpu.SemaphoreType.DMA,
            pltpu.SemaphoreType.DMA((2,)),
        ],
    )
    def sk(msg_hbm, i0_hbm, i1_hbm, z_hbm, acc_hbm, idxv, mbuf, accsh,
           spi, spm, ss):
        ci = lax.axis_index("c")
        si = lax.axis_index("s")
        r0 = si * rpt

        def zbody(gi, carry):
            r = r0 + gi * chunk
            pltpu.sync_copy(z_hbm.at[pl.ds(r, chunk)], mbuf.at[0])
            pltpu.sync_copy(mbuf.at[0], accsh.at[pl.ds(r, chunk)])
            return carry

        lax.fori_loop(0, rch, zbody, 0)
        plsc.subcore_barrier()

        def run(idx_hbm):
            ebase = si * ept

            def pwait(p):
                pltpu.make_async_copy(idx_hbm.at[pl.ds(0, chunk)],
                                      idxv.at[p], spi).wait()
                pltpu.make_async_copy(msg_hbm.at[0, pl.ds(0, chunk)],
                                      mbuf.at[p], spm).wait()

            def prefetch(g, p):
                off = ebase + g * chunk
                pltpu.async_copy(idx_hbm.at[pl.ds(off, chunk)], idxv.at[p], spi)
                pltpu.async_copy(msg_hbm.at[ci, pl.ds(off, chunk)],
                                 mbuf.at[p], spm)

            prefetch(0, 0)

            def body(g, carry):
                p = lax.rem(g, 2)
                pn = 1 - p
                pwait(p)
                pltpu.async_copy(mbuf.at[p], accsh.at[idxv.at[p]], ss.at[p],
                                 add=True)

                @pl.when(g >= 1)
                def _():
                    pltpu.make_async_copy(mbuf.at[pn],
                                          accsh.at[pl.ds(0, chunk)],
                                          ss.at[pn]).wait()

                @pl.when(g + 1 < nch)
                def _():
                    prefetch(g + 1, pn)
                return carry

            lax.fori_loop(0, nch, body, 0)
            pltpu.make_async_copy(mbuf.at[0], accsh.at[pl.ds(0, chunk)],
                                  ss.at[lax.rem(nch - 1, 2)]).wait()

        @pl.when(ci == 0)
        def _():
            run(i0_hbm)

        @pl.when(ci == 1)
        def _():
            run(i1_hbm)

        plsc.subcore_barrier()

        def obody(gi, carry):
            p = lax.rem(gi, 2)
            r = r0 + gi * chunk

            @pl.when(gi >= 2)
            def _():
                pltpu.make_async_copy(mbuf.at[p],
                                      acc_hbm.at[0, pl.ds(0, chunk)],
                                      spm).wait()

            pltpu.sync_copy(accsh.at[pl.ds(r, chunk)], mbuf.at[p])
            pltpu.async_copy(mbuf.at[p], acc_hbm.at[ci, pl.ds(r, chunk)], spm)
            return carry

        lax.fori_loop(0, rch, obody, 0)
        pltpu.make_async_copy(mbuf.at[0], acc_hbm.at[0, pl.ds(0, chunk)],
                              spm).wait()
        pltpu.make_async_copy(mbuf.at[1], acc_hbm.at[0, pl.ds(0, chunk)],
                              spm).wait()

    return sk


def kernel(x, edge_attr, edge_index, params):
    p = params
    n = x.shape[0]
    e_total = edge_attr.shape[0]
    row = edge_index[0]
    col = edge_index[1]
    _g = 80 * _SC_NS
    n_pad = ((n + _g - 1) // _g) * _g
    zeros = jnp.zeros((n_pad, _H), F32)

    def rb(v):
        return v.reshape(1, -1)

    h = _mlp2(x, p['W11_w'].T, rb(p['W11_b']), rb(p['ln1_g']), rb(p['ln1_b']),
              p['W12_w'].T, rb(p['W12_b']), blk=400)

    ha = 163840
    halves = (ha, e_total - ha)
    rows = (row[:ha], row[ha:])
    cols = (col[:ha], col[ha:])
    es = [
        _mlp2(ea, p['W21_w'].T, rb(p['W21_b']), rb(p['ln2_g']),
              rb(p['ln2_b']), p['W22_w'].T, rb(p['W22_b']), blk=1280,
              bf16_dot2=True, out_dtype=jnp.bfloat16)
        for ea in (edge_attr[:ha], edge_attr[ha:])
    ]

    gather_t = [_make_gather2(n, 2 * _H, sz, 80, jnp.uint32) for sz in halves]
    gather_h = [_make_gather2(n, _H, sz, 80, F32) for sz in halves]
    scatter = [_make_scatter2(n_pad, _H, sz, 80) for sz in halves]

    for l in range(3):
        wn = jnp.concatenate(
            [p[f'L{l}_A1_w'].T]
            + [p[f'L{l}_{nm}_w'].T[:, _ILV] for nm in ('A2', 'A3', 'B2', 'B3')],
            axis=1)
        bn = jnp.concatenate(
            [p[f'L{l}_A1_b']]
            + [p[f'L{l}_{nm}_b'][_ILV] for nm in ('A2', 'A3', 'B2', 'B3')]
        ).reshape(1, -1)
        a1h, t = _node_linear(h, wn, bn, blk=400)
        tp = _pack32(t)
        uv = [gather_t[a](tp, tp, rows[a], cols[a]) for a in range(2)]
        accs = []
        for a in range(2):
            es[a], msg = _edge_layer(es[a], uv[a][0], uv[a][1],
                                     p[f'L{l}_B1_w'].T, rb(p[f'L{l}_B1_b']),
                                     rb(p[f'L{l}_bne_g']),
                                     rb(p[f'L{l}_bne_b']), blk=1280)
            accs.append(scatter[a](msg, cols[a], rows[a], zeros))
        h = _node_update(h, a1h, accs[0], accs[1], rb(p[f'L{l}_bnh_g']),
                         rb(p[f'L{l}_bnh_b']), blk=400)

    s1t = p['s1_w'].T
    outs = []
    for a in range(2):
        hrow, hcol = gather_h[a](h, h, rows[a], cols[a])
        outs.append(_score(hrow, hcol, es[a],
                           s1t[0:_H], s1t[_H:2 * _H], s1t[2 * _H:3 * _H],
                           rb(p['s1_b']), p['s2_w'].T,
                           p['s2_b'].reshape(1, 1), blk=1280))
    return jnp.concatenate(outs, axis=0)


# speedup vs baseline: 1.0482x; 1.0482x over previous
import functools

import jax
import jax.numpy as jnp
import numpy as np
from jax import lax
from jax.experimental import pallas as pl
from jax.experimental.pallas import tpu as pltpu
from jax.experimental.pallas import tpu_sc as plsc

F32 = jnp.float32
_H = 128
_SC_NC = 2
_SC_NS = 16
_NW = _SC_NC * _SC_NS

_ILV = np.empty(_H, np.int32)
_ILV[0::2] = np.arange(_H // 2)
_ILV[1::2] = np.arange(_H // 2, _H)


def _pack32(t):
    return lax.bitcast_convert_type(
        t.reshape(t.shape[0], t.shape[1] // 2, 2), jnp.uint32)


def _unpack32(w):
    lo = lax.bitcast_convert_type(w << 16, F32)
    hi = lax.bitcast_convert_type(w & jnp.uint32(0xFFFF0000), F32)
    return lo, hi


def _ln_rows(t, g, b, eps=1e-5):
    m = jnp.mean(t, axis=-1, keepdims=True)
    v = jnp.mean((t - m) * (t - m), axis=-1, keepdims=True)
    return (t - m) * lax.rsqrt(v + eps) * g + b


def _mlp2_body(x_ref, w1_ref, b1_ref, g_ref, bb_ref, w2_ref, b2_ref, o_ref,
               *, bf16_dot2):
    t = jnp.dot(x_ref[...], w1_ref[...], preferred_element_type=F32) + b1_ref[...]
    t = jnp.maximum(t, 0.0)
    t = _ln_rows(t, g_ref[...], bb_ref[...])
    if bf16_dot2:
        o = jnp.dot(t.astype(jnp.bfloat16), w2_ref[...].astype(jnp.bfloat16),
                    preferred_element_type=F32) + b2_ref[...]
    else:
        o = jnp.dot(t, w2_ref[...], preferred_element_type=F32) + b2_ref[...]
    o_ref[...] = o.astype(o_ref.dtype)


def _mlp2(x, w1t, b1, g, bb, w2t, b2, blk, bf16_dot2=False, out_dtype=F32):
    n, fin = x.shape
    grid = (pl.cdiv(n, blk),)
    return pl.pallas_call(
        functools.partial(_mlp2_body, bf16_dot2=bf16_dot2),
        grid=grid,
        in_specs=[
            pl.BlockSpec((blk, fin), lambda i: (i, 0)),
            pl.BlockSpec((fin, _H), lambda i: (0, 0)),
            pl.BlockSpec((1, _H), lambda i: (0, 0)),
            pl.BlockSpec((1, _H), lambda i: (0, 0)),
            pl.BlockSpec((1, _H), lambda i: (0, 0)),
            pl.BlockSpec((_H, _H), lambda i: (0, 0)),
            pl.BlockSpec((1, _H), lambda i: (0, 0)),
        ],
        out_specs=pl.BlockSpec((blk, _H), lambda i: (i, 0)),
        out_shape=jax.ShapeDtypeStruct((n, _H), out_dtype),
        compiler_params=pltpu.CompilerParams(dimension_semantics=("parallel",)),
    )(x, w1t, b1, g, bb, w2t, b2)


def _node_linear_body(h_ref, w_ref, b_ref, a1_ref, t_ref):
    hw = jnp.dot(h_ref[...], w_ref[...], preferred_element_type=F32) + b_ref[...]
    a1_ref[...] = hw[:, 0 * _H:1 * _H]
    t_ref[...] = hw[:, 1 * _H:5 * _H].astype(jnp.bfloat16)


def _node_linear(h, w, b, blk):
    n = h.shape[0]
    grid = (pl.cdiv(n, blk),)
    return pl.pallas_call(
        _node_linear_body,
        grid=grid,
        in_specs=[
            pl.BlockSpec((blk, _H), lambda i: (i, 0)),
            pl.BlockSpec((_H, 5 * _H), lambda i: (0, 0)),
            pl.BlockSpec((1, 5 * _H), lambda i: (0, 0)),
        ],
        out_specs=[
            pl.BlockSpec((blk, _H), lambda i: (i, 0)),
            pl.BlockSpec((blk, 4 * _H), lambda i: (i, 0)),
        ],
        out_shape=[
            jax.ShapeDtypeStruct((n, _H), F32),
            jax.ShapeDtypeStruct((n, 4 * _H), jnp.bfloat16),
        ],
        compiler_params=pltpu.CompilerParams(dimension_semantics=("parallel",)),
    )(h, w, b)


def _edge_layer_body(e_ref, ur_ref, vc_ref, w_ref, wb_ref, g_ref, b_ref,
                     eo_ref, m_ref):
    e = e_ref[...].astype(F32)
    q = _H // 2
    ulo, uhi = _unpack32(ur_ref[...])
    vlo, vhi = _unpack32(vc_ref[...])

    def piece(lo, hi, gidx):
        return jnp.concatenate(
            [lo[:, gidx * q:(gidx + 1) * q], hi[:, gidx * q:(gidx + 1) * q]],
            axis=1)

    a2r = piece(ulo, uhi, 0)
    b2r = piece(ulo, uhi, 2)
    b3r = piece(ulo, uhi, 3)
    a3c = piece(vlo, vhi, 1)
    b2c = piece(vlo, vhi, 2)
    b3c = piece(vlo, vhi, 3)
    b1h = jnp.dot(e_ref[...], w_ref[...].astype(jnp.bfloat16),
                  preferred_element_type=F32) + wb_ref[...]
    g = g_ref[...]
    b = b_ref[...]

    def gate(t):
        t = jnp.maximum(t, 0.0)
        t = _ln_rows(t, g, b)
        t = e + t
        return t, jax.nn.sigmoid(t)

    e_ji, s_ji = gate(b1h + b2r + b3c)
    e_ik, s_ik = gate(b1h + b2c + b3r)
    m_ji = a2r * s_ji * (1.0 / (jnp.sum(s_ji, axis=1, keepdims=True) + 1e-6))
    m_ik = a3c * s_ik * (1.0 / (jnp.sum(s_ik, axis=1, keepdims=True) + 1e-6))
    eo_ref[...] = e_ji.astype(jnp.bfloat16)
    m_ref[0, :, :] = m_ji
    m_ref[1, :, :] = m_ik


def _edge_layer(e, ur, vc, w, wb, g, b, blk):
    n = e.shape[0]
    grid = (pl.cdiv(n, blk),)
    return pl.pallas_call(
        _edge_layer_body,
        grid=grid,
        in_specs=[
            pl.BlockSpec((blk, _H), lambda i: (i, 0)),
            pl.BlockSpec((blk, 2 * _H), lambda i: (i, 0)),
            pl.BlockSpec((blk, 2 * _H), lambda i: (i, 0)),
            pl.BlockSpec((_H, _H), lambda i: (0, 0)),
            pl.BlockSpec((1, _H), lambda i: (0, 0)),
            pl.BlockSpec((1, _H), lambda i: (0, 0)),
            pl.BlockSpec((1, _H), lambda i: (0, 0)),
        ],
        out_specs=[
            pl.BlockSpec((blk, _H), lambda i: (i, 0)),
            pl.BlockSpec((2, blk, _H), lambda i: (0, i, 0)),
        ],
        out_shape=[
            jax.ShapeDtypeStruct((n, _H), jnp.bfloat16),
            jax.ShapeDtypeStruct((2, n, _H), F32),
        ],
        compiler_params=pltpu.CompilerParams(dimension_semantics=("parallel",)),
    )(e, ur, vc, w, wb, g, b)


def _node_update_body(h_ref, a1_ref, acca_ref, accb_ref, g_ref, b_ref, o_ref):
    t = (a1_ref[...] + acca_ref[0, :, :] + acca_ref[1, :, :]
         + accb_ref[0, :, :] + accb_ref[1, :, :])
    t = jnp.maximum(t, 0.0)
    t = _ln_rows(t, g_ref[...], b_ref[...])
    o_ref[...] = h_ref[...] + t


def _node_update(h, a1h, acca, accb, g, b, blk):
    n = h.shape[0]
    grid = (pl.cdiv(n, blk),)
    return pl.pallas_call(
        _node_update_body,
        grid=grid,
        in_specs=[
            pl.BlockSpec((blk, _H), lambda i: (i, 0)),
            pl.BlockSpec((blk, _H), lambda i: (i, 0)),
            pl.BlockSpec((2, blk, _H), lambda i: (0, i, 0)),
            pl.BlockSpec((2, blk, _H), lambda i: (0, i, 0)),
            pl.BlockSpec((1, _H), lambda i: (0, 0)),
            pl.BlockSpec((1, _H), lambda i: (0, 0)),
        ],
        out_specs=pl.BlockSpec((blk, _H), lambda i: (i, 0)),
        out_shape=jax.ShapeDtypeStruct((n, _H), F32),
        compiler_params=pltpu.CompilerParams(dimension_semantics=("parallel",)),
    )(h, a1h, acca, accb, g, b)


def _score_body(hr_ref, hc_ref, e_ref, wa_ref, wb_ref, wc_ref, b1_ref,
                w2_ref, b2_ref, o_ref):
    bf = jnp.bfloat16
    t = (jnp.dot(hr_ref[...].astype(bf), wa_ref[...].astype(bf),
                 preferred_element_type=F32)
         + jnp.dot(hc_ref[...].astype(bf), wb_ref[...].astype(bf),
                   preferred_element_type=F32)
         + jnp.dot(e_ref[...].astype(bf), wc_ref[...].astype(bf),
                   preferred_element_type=F32)
         + b1_ref[...])
    t = jnp.maximum(t, 0.0)
    o_ref[...] = jnp.dot(t, w2_ref[...], preferred_element_type=F32) + b2_ref[...]


def _score(hr, hc, e, wa, wb, wc, b1, w2, b2, blk):
    n = hr.shape[0]
    grid = (pl.cdiv(n, blk),)
    return pl.pallas_call(
        _score_body,
        grid=grid,
        in_specs=[
            pl.BlockSpec((blk, _H), lambda i: (i, 0)),
            pl.BlockSpec((blk, _H), lambda i: (i, 0)),
            pl.BlockSpec((blk, _H), lambda i: (i, 0)),
            pl.BlockSpec((_H, _H), lambda i: (0, 0)),
            pl.BlockSpec((_H, _H), lambda i: (0, 0)),
            pl.BlockSpec((_H, _H), lambda i: (0, 0)),
            pl.BlockSpec((1, _H), lambda i: (0, 0)),
            pl.BlockSpec((_H, 1), lambda i: (0, 0)),
            pl.BlockSpec((1, 1), lambda i: (0, 0)),
        ],
        out_specs=pl.BlockSpec((blk, 1), lambda i: (i, 0)),
        out_shape=jax.ShapeDtypeStruct((n, 1), F32),
        compiler_params=pltpu.CompilerParams(dimension_semantics=("parallel",)),
    )(hr, hc, e, wa, wb, wc, b1, w2, b2)


def _make_gather2(n_rows, d, e_total, chunk, dtype=F32):
    epw = e_total // _NW
    nch = epw // chunk
    mesh = plsc.VectorSubcoreMesh(core_axis_name="c", subcore_axis_name="s")

    @functools.partial(
        pl.kernel,
        mesh=mesh,
        out_type=[jax.ShapeDtypeStruct((e_total, d), dtype),
                  jax.ShapeDtypeStruct((e_total, d), dtype)],
        scratch_types=[
            pltpu.VMEM((epw,), jnp.int32),
            pltpu.VMEM((epw,), jnp.int32),
            pltpu.VMEM((2, chunk, d), dtype),
            pltpu.VMEM((2, chunk, d), dtype),
            pltpu.SemaphoreType.DMA,
            pltpu.SemaphoreType.DMA,
            pltpu.SemaphoreType.DMA,
            pltpu.SemaphoreType.DMA,
        ],
    )
    def gk(t0_hbm, t1_hbm, i0_hbm, i1_hbm, o0_hbm, o1_hbm,
           i0v, i1v, b0, b1, sg0, sg1, sw0, sw1):
        wid = lax.axis_index("s") * _SC_NC + lax.axis_index("c")
        base = wid * epw

        def gwait(p):
            pltpu.make_async_copy(t0_hbm.at[pl.ds(0, chunk)], b0.at[p], sg0).wait()
            pltpu.make_async_copy(t1_hbm.at[pl.ds(0, chunk)], b1.at[p], sg1).wait()

        def wwait(p):
            pltpu.make_async_copy(b0.at[p], o0_hbm.at[pl.ds(0, chunk)], sw0).wait()
            pltpu.make_async_copy(b1.at[p], o1_hbm.at[pl.ds(0, chunk)], sw1).wait()

        def gissue(g, p):
            pltpu.async_copy(t0_hbm.at[i0v.at[pl.ds(g * chunk, chunk)]],
                             b0.at[p], sg0)
            pltpu.async_copy(t1_hbm.at[i1v.at[pl.ds(g * chunk, chunk)]],
                             b1.at[p], sg1)

        pltpu.sync_copy(i0_hbm.at[pl.ds(base, epw)], i0v)
        pltpu.sync_copy(i1_hbm.at[pl.ds(base, epw)], i1v)
        gissue(0, 0)

        def body(g, carry):
            p = lax.rem(g, 2)
            pn = 1 - p
            off = base + g * chunk
            gwait(p)
            pltpu.async_copy(b0.at[p], o0_hbm.at[pl.ds(off, chunk)], sw0)
            pltpu.async_copy(b1.at[p], o1_hbm.at[pl.ds(off, chunk)], sw1)

            @pl.when(g + 1 < nch)
            def _():
                @pl.when(g >= 1)
                def _():
                    wwait(pn)

                gissue(g + 1, pn)
            return carry

        lax.fori_loop(0, nch, body, 0)
        wwait(0)
        wwait(1)

    return gk


def _make_scatter2(n_rows, d, e_total, chunk):
    ept = e_total // _SC_NS
    nch = ept // chunk
    rpt = n_rows // _SC_NS
    assert rpt % chunk == 0 and chunk % 8 == 0 and n_rows % _SC_NS == 0
    rch = rpt // chunk
    mesh = plsc.VectorSubcoreMesh(core_axis_name="c", subcore_axis_name="s")

    @functools.partial(
        pl.kernel,
        mesh=mesh,
        out_type=jax.ShapeDtypeStruct((2, n_rows, d), F32),
        scratch_types=[
            pltpu.VMEM((2, chunk), jnp.int32),
            pltpu.VMEM((2, chunk, d), F32),
            pltpu.VMEM_SHARED((n_rows, d), F32),
            pltpu.SemaphoreType.DMA,
            pltpu.SemaphoreType.DMA,
            pltpu.SemaphoreType.DMA((2,)),
        ],
    )
    def sk(msg_hbm, i0_hbm, i1_hbm, z_hbm, acc_hbm, idxv, mbuf, accsh,
           spi, spm, ss):
        ci = lax.axis_index("c")
        si = lax.axis_index("s")
        r0 = si * rpt

        def zbody(gi, carry):
            r = r0 + gi * chunk
            pltpu.sync_copy(z_hbm.at[pl.ds(r, chunk)], mbuf.at[0])
            pltpu.sync_copy(mbuf.at[0], accsh.at[pl.ds(r, chunk)])
            return carry

        lax.fori_loop(0, rch, zbody, 0)
        plsc.subcore_barrier()

        def run(idx_hbm):
            ebase = si * ept

            def pwait(p):
                pltpu.make_async_copy(idx_hbm.at[pl.ds(0, chunk)],
                                      idxv.at[p], spi).wait()
                pltpu.make_async_copy(msg_hbm.at[0, pl.ds(0, chunk)],
                                      mbuf.at[p], spm).wait()

            def prefetch(g, p):
                off = ebase + g * chunk
                pltpu.async_copy(idx_hbm.at[pl.ds(off, chunk)], idxv.at[p], spi)
                pltpu.async_copy(msg_hbm.at[ci, pl.ds(off, chunk)],
                                 mbuf.at[p], spm)

            prefetch(0, 0)

            def body(g, carry):
                p = lax.rem(g, 2)
                pn = 1 - p
                pwait(p)
                pltpu.async_copy(mbuf.at[p], accsh.at[idxv.at[p]], ss.at[p],
                                 add=True)

                @pl.when(g >= 1)
                def _():
                    pltpu.make_async_copy(mbuf.at[pn],
                                          accsh.at[pl.ds(0, chunk)],
                                          ss.at[pn]).wait()

                @pl.when(g + 1 < nch)
                def _():
                    prefetch(g + 1, pn)
                return carry

            lax.fori_loop(0, nch, body, 0)
            pltpu.make_async_copy(mbuf.at[0], accsh.at[pl.ds(0, chunk)],
                                  ss.at[lax.rem(nch - 1, 2)]).wait()

        @pl.when(ci == 0)
        def _():
            run(i0_hbm)

        @pl.when(ci == 1)
        def _():
            run(i1_hbm)

        plsc.subcore_barrier()

        def obody(gi, carry):
            p = lax.rem(gi, 2)
            r = r0 + gi * chunk

            @pl.when(gi >= 2)
            def _():
                pltpu.make_async_copy(mbuf.at[p],
                                      acc_hbm.at[0, pl.ds(0, chunk)],
                                      spm).wait()

            pltpu.sync_copy(accsh.at[pl.ds(r, chunk)], mbuf.at[p])
            pltpu.async_copy(mbuf.at[p], acc_hbm.at[ci, pl.ds(r, chunk)], spm)
            return carry

        lax.fori_loop(0, rch, obody, 0)
        pltpu.make_async_copy(mbuf.at[0], acc_hbm.at[0, pl.ds(0, chunk)],
                              spm).wait()
        pltpu.make_async_copy(mbuf.at[1], acc_hbm.at[0, pl.ds(0, chunk)],
                              spm).wait()

    return sk


def kernel(x, edge_attr, edge_index, params):
    p = params
    n = x.shape[0]
    e_total = edge_attr.shape[0]
    row = edge_index[0]
    col = edge_index[1]
    _g = 80 * _SC_NS
    n_pad = ((n + _g - 1) // _g) * _g
    zeros = jnp.zeros((n_pad, _H), F32)

    def rb(v):
        return v.reshape(1, -1)

    h = _mlp2(x, p['W11_w'].T, rb(p['W11_b']), rb(p['ln1_g']), rb(p['ln1_b']),
              p['W12_w'].T, rb(p['W12_b']), blk=400)

    ha = 163840
    halves = (ha, e_total - ha)
    rows = (row[:ha], row[ha:])
    cols = (col[:ha], col[ha:])
    es = [
        _mlp2(ea, p['W21_w'].T, rb(p['W21_b']), rb(p['ln2_g']),
              rb(p['ln2_b']), p['W22_w'].T, rb(p['W22_b']), blk=1280,
              bf16_dot2=True, out_dtype=jnp.bfloat16)
        for ea in (edge_attr[:ha], edge_attr[ha:])
    ]

    gather_t = [_make_gather2(n, 2 * _H, sz, 80, jnp.uint32) for sz in halves]
    gather_h = [_make_gather2(n, _H, sz, 80, F32) for sz in halves]
    scatter = [_make_scatter2(n_pad, _H, sz, 80) for sz in halves]

    for l in range(3):
        wn = jnp.concatenate(
            [p[f'L{l}_A1_w'].T]
            + [p[f'L{l}_{nm}_w'].T[:, _ILV] for nm in ('A2', 'A3', 'B2', 'B3')],
            axis=1)
        bn = jnp.concatenate(
            [p[f'L{l}_A1_b']]
            + [p[f'L{l}_{nm}_b'][_ILV] for nm in ('A2', 'A3', 'B2', 'B3')]
        ).reshape(1, -1)
        a1h, t = _node_linear(h, wn, bn, blk=400)
        tp = _pack32(t)
        uv = [gather_t[a](tp, tp, rows[a], cols[a]) for a in range(2)]
        accs = []
        for a in range(2):
            es[a], msg = _edge_layer(es[a], uv[a][0], uv[a][1],
                                     p[f'L{l}_B1_w'].T, rb(p[f'L{l}_B1_b']),
                                     rb(p[f'L{l}_bne_g']),
                                     rb(p[f'L{l}_bne_b']), blk=2560)
            accs.append(scatter[a](msg, cols[a], rows[a], zeros))
        h = _node_update(h, a1h, accs[0], accs[1], rb(p[f'L{l}_bnh_g']),
                         rb(p[f'L{l}_bnh_b']), blk=400)

    s1t = p['s1_w'].T
    outs = []
    for a in range(2):
        hrow, hcol = gather_h[a](h, h, rows[a], cols[a])
        outs.append(_score(hrow, hcol, es[a],
                           s1t[0:_H], s1t[_H:2 * _H], s1t[2 * _H:3 * _H],
                           rb(p['s1_b']), p['s2_w'].T,
                           p['s2_b'].reshape(1, 1), blk=2560))
    return jnp.concatenate(outs, axis=0)


# speedup vs baseline: 1.0638x; 1.0148x over previous
import functools

import jax
import jax.numpy as jnp
import numpy as np
from jax import lax
from jax.experimental import pallas as pl
from jax.experimental.pallas import tpu as pltpu
from jax.experimental.pallas import tpu_sc as plsc

F32 = jnp.float32
_H = 128
_SC_NC = 2
_SC_NS = 16
_NW = _SC_NC * _SC_NS

_ILV = np.empty(_H, np.int32)
_ILV[0::2] = np.arange(_H // 2)
_ILV[1::2] = np.arange(_H // 2, _H)


def _pack32(t):
    return lax.bitcast_convert_type(
        t.reshape(t.shape[0], t.shape[1] // 2, 2), jnp.uint32)


def _unpack32(w):
    lo = lax.bitcast_convert_type(w << 16, F32)
    hi = lax.bitcast_convert_type(w & jnp.uint32(0xFFFF0000), F32)
    return lo, hi


def _ln_rows(t, g, b, eps=1e-5):
    m = jnp.mean(t, axis=-1, keepdims=True)
    v = jnp.mean((t - m) * (t - m), axis=-1, keepdims=True)
    return (t - m) * lax.rsqrt(v + eps) * g + b


def _mlp2_body(x_ref, w1_ref, b1_ref, g_ref, bb_ref, w2_ref, b2_ref, o_ref,
               *, bf16_dot2):
    t = jnp.dot(x_ref[...], w1_ref[...], preferred_element_type=F32) + b1_ref[...]
    t = jnp.maximum(t, 0.0)
    t = _ln_rows(t, g_ref[...], bb_ref[...])
    if bf16_dot2:
        o = jnp.dot(t.astype(jnp.bfloat16), w2_ref[...].astype(jnp.bfloat16),
                    preferred_element_type=F32) + b2_ref[...]
    else:
        o = jnp.dot(t, w2_ref[...], preferred_element_type=F32) + b2_ref[...]
    o_ref[...] = o.astype(o_ref.dtype)


def _mlp2(x, w1t, b1, g, bb, w2t, b2, blk, bf16_dot2=False, out_dtype=F32):
    n, fin = x.shape
    grid = (pl.cdiv(n, blk),)
    return pl.pallas_call(
        functools.partial(_mlp2_body, bf16_dot2=bf16_dot2),
        grid=grid,
        in_specs=[
            pl.BlockSpec((blk, fin), lambda i: (i, 0)),
            pl.BlockSpec((fin, _H), lambda i: (0, 0)),
            pl.BlockSpec((1, _H), lambda i: (0, 0)),
            pl.BlockSpec((1, _H), lambda i: (0, 0)),
            pl.BlockSpec((1, _H), lambda i: (0, 0)),
            pl.BlockSpec((_H, _H), lambda i: (0, 0)),
            pl.BlockSpec((1, _H), lambda i: (0, 0)),
        ],
        out_specs=pl.BlockSpec((blk, _H), lambda i: (i, 0)),
        out_shape=jax.ShapeDtypeStruct((n, _H), out_dtype),
        compiler_params=pltpu.CompilerParams(dimension_semantics=("parallel",)),
    )(x, w1t, b1, g, bb, w2t, b2)


def _node_linear_body(h_ref, w_ref, b_ref, a1_ref, t_ref):
    hw = jnp.dot(h_ref[...], w_ref[...], preferred_element_type=F32) + b_ref[...]
    a1_ref[...] = hw[:, 0 * _H:1 * _H]
    t_ref[...] = hw[:, 1 * _H:5 * _H].astype(jnp.bfloat16)


def _node_linear(h, w, b, blk):
    n = h.shape[0]
    grid = (pl.cdiv(n, blk),)
    return pl.pallas_call(
        _node_linear_body,
        grid=grid,
        in_specs=[
            pl.BlockSpec((blk, _H), lambda i: (i, 0)),
            pl.BlockSpec((_H, 5 * _H), lambda i: (0, 0)),
            pl.BlockSpec((1, 5 * _H), lambda i: (0, 0)),
        ],
        out_specs=[
            pl.BlockSpec((blk, _H), lambda i: (i, 0)),
            pl.BlockSpec((blk, 4 * _H), lambda i: (i, 0)),
        ],
        out_shape=[
            jax.ShapeDtypeStruct((n, _H), F32),
            jax.ShapeDtypeStruct((n, 4 * _H), jnp.bfloat16),
        ],
        compiler_params=pltpu.CompilerParams(dimension_semantics=("parallel",)),
    )(h, w, b)


def _edge_layer_body(e_ref, ur_ref, vc_ref, w_ref, wb_ref, g_ref, b_ref,
                     eo_ref, m_ref):
    e = e_ref[...].astype(F32)
    q = _H // 2
    ulo, uhi = _unpack32(ur_ref[...])
    vlo, vhi = _unpack32(vc_ref[...])

    def piece(lo, hi, gidx):
        return jnp.concatenate(
            [lo[:, gidx * q:(gidx + 1) * q], hi[:, gidx * q:(gidx + 1) * q]],
            axis=1)

    a2r = piece(ulo, uhi, 0)
    b2r = piece(ulo, uhi, 2)
    b3r = piece(ulo, uhi, 3)
    a3c = piece(vlo, vhi, 1)
    b2c = piece(vlo, vhi, 2)
    b3c = piece(vlo, vhi, 3)
    b1h = jnp.dot(e_ref[...], w_ref[...].astype(jnp.bfloat16),
                  preferred_element_type=F32) + wb_ref[...]
    g = g_ref[...]
    b = b_ref[...]

    def gate(t):
        t = jnp.maximum(t, 0.0)
        t = _ln_rows(t, g, b)
        t = e + t
        return t, jax.nn.sigmoid(t)

    e_ji, s_ji = gate(b1h + b2r + b3c)
    e_ik, s_ik = gate(b1h + b2c + b3r)
    m_ji = a2r * s_ji * (1.0 / (jnp.sum(s_ji, axis=1, keepdims=True) + 1e-6))
    m_ik = a3c * s_ik * (1.0 / (jnp.sum(s_ik, axis=1, keepdims=True) + 1e-6))
    eo_ref[...] = e_ji.astype(jnp.bfloat16)
    m_ref[0, :, :] = m_ji
    m_ref[1, :, :] = m_ik


def _edge_layer(e, ur, vc, w, wb, g, b, blk):
    n = e.shape[0]
    grid = (pl.cdiv(n, blk),)
    return pl.pallas_call(
        _edge_layer_body,
        grid=grid,
        in_specs=[
            pl.BlockSpec((blk, _H), lambda i: (i, 0)),
            pl.BlockSpec((blk, 2 * _H), lambda i: (i, 0)),
            pl.BlockSpec((blk, 2 * _H), lambda i: (i, 0)),
            pl.BlockSpec((_H, _H), lambda i: (0, 0)),
            pl.BlockSpec((1, _H), lambda i: (0, 0)),
            pl.BlockSpec((1, _H), lambda i: (0, 0)),
            pl.BlockSpec((1, _H), lambda i: (0, 0)),
        ],
        out_specs=[
            pl.BlockSpec((blk, _H), lambda i: (i, 0)),
            pl.BlockSpec((2, blk, _H), lambda i: (0, i, 0)),
        ],
        out_shape=[
            jax.ShapeDtypeStruct((n, _H), jnp.bfloat16),
            jax.ShapeDtypeStruct((2, n, _H), F32),
        ],
        compiler_params=pltpu.CompilerParams(dimension_semantics=("parallel",)),
    )(e, ur, vc, w, wb, g, b)


def _node_update_body(h_ref, a1_ref, acca_ref, accb_ref, g_ref, b_ref, o_ref):
    t = (a1_ref[...] + acca_ref[0, :, :] + acca_ref[1, :, :]
         + accb_ref[0, :, :] + accb_ref[1, :, :])
    t = jnp.maximum(t, 0.0)
    t = _ln_rows(t, g_ref[...], b_ref[...])
    o_ref[...] = h_ref[...] + t


def _node_update(h, a1h, acca, accb, g, b, blk):
    n = h.shape[0]
    grid = (pl.cdiv(n, blk),)
    return pl.pallas_call(
        _node_update_body,
        grid=grid,
        in_specs=[
            pl.BlockSpec((blk, _H), lambda i: (i, 0)),
            pl.BlockSpec((blk, _H), lambda i: (i, 0)),
            pl.BlockSpec((2, blk, _H), lambda i: (0, i, 0)),
            pl.BlockSpec((2, blk, _H), lambda i: (0, i, 0)),
            pl.BlockSpec((1, _H), lambda i: (0, 0)),
            pl.BlockSpec((1, _H), lambda i: (0, 0)),
        ],
        out_specs=pl.BlockSpec((blk, _H), lambda i: (i, 0)),
        out_shape=jax.ShapeDtypeStruct((n, _H), F32),
        compiler_params=pltpu.CompilerParams(dimension_semantics=("parallel",)),
    )(h, a1h, acca, accb, g, b)


def _score_body(hr_ref, hc_ref, e_ref, wa_ref, wb_ref, wc_ref, b1_ref,
                w2_ref, b2_ref, o_ref):
    bf = jnp.bfloat16
    t = (jnp.dot(hr_ref[...].astype(bf), wa_ref[...].astype(bf),
                 preferred_element_type=F32)
         + jnp.dot(hc_ref[...].astype(bf), wb_ref[...].astype(bf),
                   preferred_element_type=F32)
         + jnp.dot(e_ref[...].astype(bf), wc_ref[...].astype(bf),
                   preferred_element_type=F32)
         + b1_ref[...])
    t = jnp.maximum(t, 0.0)
    o_ref[...] = jnp.dot(t, w2_ref[...], preferred_element_type=F32) + b2_ref[...]


def _score(hr, hc, e, wa, wb, wc, b1, w2, b2, blk):
    n = hr.shape[0]
    grid = (pl.cdiv(n, blk),)
    return pl.pallas_call(
        _score_body,
        grid=grid,
        in_specs=[
            pl.BlockSpec((blk, _H), lambda i: (i, 0)),
            pl.BlockSpec((blk, _H), lambda i: (i, 0)),
            pl.BlockSpec((blk, _H), lambda i: (i, 0)),
            pl.BlockSpec((_H, _H), lambda i: (0, 0)),
            pl.BlockSpec((_H, _H), lambda i: (0, 0)),
            pl.BlockSpec((_H, _H), lambda i: (0, 0)),
            pl.BlockSpec((1, _H), lambda i: (0, 0)),
            pl.BlockSpec((_H, 1), lambda i: (0, 0)),
            pl.BlockSpec((1, 1), lambda i: (0, 0)),
        ],
        out_specs=pl.BlockSpec((blk, 1), lambda i: (i, 0)),
        out_shape=jax.ShapeDtypeStruct((n, 1), F32),
        compiler_params=pltpu.CompilerParams(dimension_semantics=("parallel",)),
    )(hr, hc, e, wa, wb, wc, b1, w2, b2)


def _make_gather2(n_rows, d, e_total, chunk, dtype=F32):
    epw = e_total // _NW
    nch = epw // chunk
    mesh = plsc.VectorSubcoreMesh(core_axis_name="c", subcore_axis_name="s")

    @functools.partial(
        pl.kernel,
        mesh=mesh,
        out_type=[jax.ShapeDtypeStruct((e_total, d), dtype),
                  jax.ShapeDtypeStruct((e_total, d), dtype)],
        scratch_types=[
            pltpu.VMEM((epw,), jnp.int32),
            pltpu.VMEM((epw,), jnp.int32),
            pltpu.VMEM((2, chunk, d), dtype),
            pltpu.VMEM((2, chunk, d), dtype),
            pltpu.SemaphoreType.DMA,
            pltpu.SemaphoreType.DMA,
            pltpu.SemaphoreType.DMA,
            pltpu.SemaphoreType.DMA,
        ],
    )
    def gk(t0_hbm, t1_hbm, i0_hbm, i1_hbm, o0_hbm, o1_hbm,
           i0v, i1v, b0, b1, sg0, sg1, sw0, sw1):
        wid = lax.axis_index("s") * _SC_NC + lax.axis_index("c")
        base = wid * epw

        def gwait(p):
            pltpu.make_async_copy(t0_hbm.at[pl.ds(0, chunk)], b0.at[p], sg0).wait()
            pltpu.make_async_copy(t1_hbm.at[pl.ds(0, chunk)], b1.at[p], sg1).wait()

        def wwait(p):
            pltpu.make_async_copy(b0.at[p], o0_hbm.at[pl.ds(0, chunk)], sw0).wait()
            pltpu.make_async_copy(b1.at[p], o1_hbm.at[pl.ds(0, chunk)], sw1).wait()

        def gissue(g, p):
            pltpu.async_copy(t0_hbm.at[i0v.at[pl.ds(g * chunk, chunk)]],
                             b0.at[p], sg0)
            pltpu.async_copy(t1_hbm.at[i1v.at[pl.ds(g * chunk, chunk)]],
                             b1.at[p], sg1)

        pltpu.sync_copy(i0_hbm.at[pl.ds(base, epw)], i0v)
        pltpu.sync_copy(i1_hbm.at[pl.ds(base, epw)], i1v)
        gissue(0, 0)

        def body(g, carry):
            p = lax.rem(g, 2)
            pn = 1 - p
            off = base + g * chunk
            gwait(p)
            pltpu.async_copy(b0.at[p], o0_hbm.at[pl.ds(off, chunk)], sw0)
            pltpu.async_copy(b1.at[p], o1_hbm.at[pl.ds(off, chunk)], sw1)

            @pl.when(g + 1 < nch)
            def _():
                @pl.when(g >= 1)
                def _():
                    wwait(pn)

                gissue(g + 1, pn)
            return carry

        lax.fori_loop(0, nch, body, 0)
        wwait(0)
        wwait(1)

    return gk


def _make_scatter2(n_rows, d, e_total, chunk):
    ept = e_total // _SC_NS
    nch = ept // chunk
    rpt = n_rows // _SC_NS
    assert rpt % chunk == 0 and chunk % 8 == 0 and n_rows % _SC_NS == 0
    rch = rpt // chunk
    mesh = plsc.VectorSubcoreMesh(core_axis_name="c", subcore_axis_name="s")

    @functools.partial(
        pl.kernel,
        mesh=mesh,
        out_type=jax.ShapeDtypeStruct((2, n_rows, d), F32),
        scratch_types=[
            pltpu.VMEM((2, chunk), jnp.int32),
            pltpu.VMEM((2, chunk, d), F32),
            pltpu.VMEM_SHARED((n_rows, d), F32),
            pltpu.SemaphoreType.DMA,
            pltpu.SemaphoreType.DMA,
            pltpu.SemaphoreType.DMA((2,)),
        ],
    )
    def sk(msg_hbm, i0_hbm, i1_hbm, z_hbm, acc_hbm, idxv, mbuf, accsh,
           spi, spm, ss):
        ci = lax.axis_index("c")
        si = lax.axis_index("s")
        r0 = si * rpt

        def zbody(gi, carry):
            r = r0 + gi * chunk
            pltpu.sync_copy(z_hbm.at[pl.ds(r, chunk)], mbuf.at[0])
            pltpu.sync_copy(mbuf.at[0], accsh.at[pl.ds(r, chunk)])
            return carry

        lax.fori_loop(0, rch, zbody, 0)
        plsc.subcore_barrier()

        def run(idx_hbm):
            ebase = si * ept

            def pwait(p):
                pltpu.make_async_copy(idx_hbm.at[pl.ds(0, chunk)],
                                      idxv.at[p], spi).wait()
                pltpu.make_async_copy(msg_hbm.at[0, pl.ds(0, chunk)],
                                      mbuf.at[p], spm).wait()

            def prefetch(g, p):
                off = ebase + g * chunk
                pltpu.async_copy(idx_hbm.at[pl.ds(off, chunk)], idxv.at[p], spi)
                pltpu.async_copy(msg_hbm.at[ci, pl.ds(off, chunk)],
                                 mbuf.at[p], spm)

            prefetch(0, 0)

            def body(g, carry):
                p = lax.rem(g, 2)
                pn = 1 - p
                pwait(p)
                pltpu.async_copy(mbuf.at[p], accsh.at[idxv.at[p]], ss.at[p],
                                 add=True)

                @pl.when(g >= 1)
                def _():
                    pltpu.make_async_copy(mbuf.at[pn],
                                          accsh.at[pl.ds(0, chunk)],
                                          ss.at[pn]).wait()

                @pl.when(g + 1 < nch)
                def _():
                    prefetch(g + 1, pn)
                return carry

            lax.fori_loop(0, nch, body, 0)
            pltpu.make_async_copy(mbuf.at[0], accsh.at[pl.ds(0, chunk)],
                                  ss.at[lax.rem(nch - 1, 2)]).wait()

        @pl.when(ci == 0)
        def _():
            run(i0_hbm)

        @pl.when(ci == 1)
        def _():
            run(i1_hbm)

        plsc.subcore_barrier()

        def obody(gi, carry):
            p = lax.rem(gi, 2)
            r = r0 + gi * chunk

            @pl.when(gi >= 2)
            def _():
                pltpu.make_async_copy(mbuf.at[p],
                                      acc_hbm.at[0, pl.ds(0, chunk)],
                                      spm).wait()

            pltpu.sync_copy(accsh.at[pl.ds(r, chunk)], mbuf.at[p])
            pltpu.async_copy(mbuf.at[p], acc_hbm.at[ci, pl.ds(r, chunk)], spm)
            return carry

        lax.fori_loop(0, rch, obody, 0)
        pltpu.make_async_copy(mbuf.at[0], acc_hbm.at[0, pl.ds(0, chunk)],
                              spm).wait()
        pltpu.make_async_copy(mbuf.at[1], acc_hbm.at[0, pl.ds(0, chunk)],
                              spm).wait()

    return sk


def kernel(x, edge_attr, edge_index, params):
    p = params
    n = x.shape[0]
    e_total = edge_attr.shape[0]
    row = edge_index[0]
    col = edge_index[1]
    _g = 80 * _SC_NS
    n_pad = ((n + _g - 1) // _g) * _g
    zeros = jnp.zeros((n_pad, _H), F32)

    def rb(v):
        return v.reshape(1, -1)

    h = _mlp2(x, p['W11_w'].T, rb(p['W11_b']), rb(p['ln1_g']), rb(p['ln1_b']),
              p['W12_w'].T, rb(p['W12_b']), blk=2000)

    ha = 163840
    halves = (ha, e_total - ha)
    rows = (row[:ha], row[ha:])
    cols = (col[:ha], col[ha:])
    es = [
        _mlp2(ea, p['W21_w'].T, rb(p['W21_b']), rb(p['ln2_g']),
              rb(p['ln2_b']), p['W22_w'].T, rb(p['W22_b']), blk=1280,
              bf16_dot2=True, out_dtype=jnp.bfloat16)
        for ea in (edge_attr[:ha], edge_attr[ha:])
    ]

    gather_t = [_make_gather2(n, 2 * _H, sz, 80, jnp.uint32) for sz in halves]
    gather_h = [_make_gather2(n, _H, sz, 80, F32) for sz in halves]
    scatter = [_make_scatter2(n_pad, _H, sz, 80) for sz in halves]

    for l in range(3):
        wn = jnp.concatenate(
            [p[f'L{l}_A1_w'].T]
            + [p[f'L{l}_{nm}_w'].T[:, _ILV] for nm in ('A2', 'A3', 'B2', 'B3')],
            axis=1)
        bn = jnp.concatenate(
            [p[f'L{l}_A1_b']]
            + [p[f'L{l}_{nm}_b'][_ILV] for nm in ('A2', 'A3', 'B2', 'B3')]
        ).reshape(1, -1)
        a1h, t = _node_linear(h, wn, bn, blk=2000)
        tp = _pack32(t)
        uv = [gather_t[a](tp, tp, rows[a], cols[a]) for a in range(2)]
        accs = []
        for a in range(2):
            es[a], msg = _edge_layer(es[a], uv[a][0], uv[a][1],
                                     p[f'L{l}_B1_w'].T, rb(p[f'L{l}_B1_b']),
                                     rb(p[f'L{l}_bne_g']),
                                     rb(p[f'L{l}_bne_b']), blk=2560)
            accs.append(scatter[a](msg, cols[a], rows[a], zeros))
        h = _node_update(h, a1h, accs[0], accs[1], rb(p[f'L{l}_bnh_g']),
                         rb(p[f'L{l}_bnh_b']), blk=2000)

    s1t = p['s1_w'].T
    outs = []
    for a in range(2):
        hrow, hcol = gather_h[a](h, h, rows[a], cols[a])
        outs.append(_score(hrow, hcol, es[a],
                           s1t[0:_H], s1t[_H:2 * _H], s1t[2 * _H:3 * _H],
                           rb(p['s1_b']), p['s2_w'].T,
                           p['s2_b'].reshape(1, 1), blk=2560))
    return jnp.concatenate(outs, axis=0)
